# Initial kernel scaffold; baseline (speedup 1.0000x reference)
import jax
import jax.numpy as jnp
from jax.experimental import pallas as pl


def kernel(x, edge_index, Wk, Wq, Wv):
    raise NotImplementedError("write your pallas kernel here")


# speedup vs baseline: 3.9659x; 3.9659x over previous
import jax
import jax.numpy as jnp
from jax import lax
from jax.experimental import pallas as pl
from jax.experimental.pallas import tpu as pltpu
from jax.experimental.pallas import tpu_sc as plsc

N = 10000
E = 320000
D = 128
NP = 10240
NC = 2
NS = 16
L = 16
NW = NC * NS
EPW = E // NW
CH = 80
NCH = EPW // CH
GP = CH // L
NR = NP // NS
NPW = NP // NW

_mesh = plsc.VectorSubcoreMesh(core_axis_name="c", subcore_axis_name="s")
_sc_params = pltpu.CompilerParams(needs_layout_passes=False)


def _kqv_body(x_ref, wk_ref, wq_ref, wv_ref, k_ref, q_ref, v_ref):
    xb = x_ref[...]
    dn = (((1,), (1,)), ((), ()))
    k_ref[...] = lax.dot_general(xb, wk_ref[...], dn,
                                 preferred_element_type=jnp.float32)
    q_ref[...] = lax.dot_general(xb, wq_ref[...], dn,
                                 preferred_element_type=jnp.float32)
    v_ref[...] = lax.dot_general(xb, wv_ref[...], dn,
                                 preferred_element_type=jnp.float32)


def _kqv(x, wk, wq, wv):
    rb = 2000
    wspec = pl.BlockSpec((D, D), lambda i: (0, 0))
    xspec = pl.BlockSpec((rb, D), lambda i: (i, 0))
    return pl.pallas_call(
        _kqv_body,
        grid=(N // rb,),
        in_specs=[xspec, wspec, wspec, wspec],
        out_specs=[xspec, xspec, xspec],
        out_shape=[jax.ShapeDtypeStruct((N, D), jnp.float32)] * 3,
    )(x, wk, wq, wv)


def _k1_body(k_hbm, q_hbm, src_hbm, dst_hbm, e_out, mpart_out,
             idxs, idxd, krows, qrows, ebuf, mpriv, sem1, sem2):
    c = lax.axis_index("c")
    s = lax.axis_index("s")
    wid = s * NC + c
    lanes = lax.broadcasted_iota(jnp.int32, (L,), 0)

    def init_body(i, carry):
        mpriv[pl.ds(i * L, L)] = jnp.full((L,), -1e30, jnp.float32)
        return carry
    lax.fori_loop(0, NP // L, init_body, 0)

    def chunk_body(ci, carry):
        base = wid * EPW + ci * CH
        pltpu.sync_copy(src_hbm.at[pl.ds(base, CH)], idxs)
        pltpu.sync_copy(dst_hbm.at[pl.ds(base, CH)], idxd)
        cpk = pltpu.async_copy(k_hbm.at[idxs], krows, sem1)
        cpq = pltpu.async_copy(q_hbm.at[idxd], qrows, sem2)
        cpk.wait()
        cpq.wait()
        for g in range(GP):
            rows = lanes + g * L

            def dot_body(j, acc):
                jv = jnp.full((L,), j, jnp.int32)
                kv = plsc.load_gather(krows, [rows, jv])
                qv = plsc.load_gather(qrows, [rows, jv])
                return acc + kv * qv

            acc = lax.fori_loop(0, D, dot_body, jnp.zeros((L,), jnp.float32))
            ev = jnp.where(acc >= 0, acc, acc * 0.01)
            ebuf[pl.ds(g * L, L)] = ev
            dstv = idxd[pl.ds(g * L, L)]

            def mx_cond(pend):
                return jnp.sum(pend.astype(jnp.int32)) > 0

            def mx_body(pend):
                cur = plsc.load_gather(mpriv, [dstv])
                need = jnp.logical_and(pend, ev > cur)
                plsc.store_scatter(mpriv, [dstv], jnp.maximum(cur, ev),
                                   mask=need)
                cur2 = plsc.load_gather(mpriv, [dstv])
                return cur2 < ev

            lax.while_loop(mx_cond, mx_body, jnp.ones((L,), jnp.bool_))
        pltpu.sync_copy(ebuf, e_out.at[pl.ds(base, CH)])
        return carry
    lax.fori_loop(0, NCH, chunk_body, 0)
    pltpu.sync_copy(mpriv, mpart_out.at[pl.ds(wid * NP, NP)])


def _k1(k, q, src, dst):
    return pl.kernel(
        _k1_body,
        out_type=(jax.ShapeDtypeStruct((E,), jnp.float32),
                  jax.ShapeDtypeStruct((NW * NP,), jnp.float32)),
        mesh=_mesh,
        scratch_types=[
            pltpu.VMEM((CH,), jnp.int32),
            pltpu.VMEM((CH,), jnp.int32),
            pltpu.VMEM((CH, D), jnp.float32),
            pltpu.VMEM((CH, D), jnp.float32),
            pltpu.VMEM((CH,), jnp.float32),
            pltpu.VMEM((NP,), jnp.float32),
            pltpu.SemaphoreType.DMA,
            pltpu.SemaphoreType.DMA,
        ],
        compiler_params=_sc_params,
    )(k, q, src, dst)


def _k2_body(mpart_in, m_out, acc, tmp):
    c = lax.axis_index("c")
    s = lax.axis_index("s")
    wid = s * NC + c
    base = wid * NPW
    pltpu.sync_copy(mpart_in.at[pl.ds(base, NPW)], acc)

    def red(j, carry):
        pltpu.sync_copy(mpart_in.at[pl.ds(j * NP + base, NPW)], tmp)
        for t in range(NPW // L):
            sl = pl.ds(t * L, L)
            acc[sl] = jnp.maximum(acc[sl], tmp[sl])
        return carry
    lax.fori_loop(1, NW, red, 0)
    pltpu.sync_copy(acc, m_out.at[pl.ds(base, NPW)])


def _k2(m_part):
    return pl.kernel(
        _k2_body,
        out_type=jax.ShapeDtypeStruct((NP,), jnp.float32),
        mesh=_mesh,
        scratch_types=[
            pltpu.VMEM((NPW,), jnp.float32),
            pltpu.VMEM((NPW,), jnp.float32),
        ],
        compiler_params=_sc_params,
    )(m_part)


def _k3_body(v_hbm, src_hbm, dst_hbm, e_hbm, m_hbm, zrows_hbm, zvec_hbm,
             nump_out, denp_out,
             mloc, idxs, idxd, ebuf, exbuf, vrows, num_s, den_s, semv):
    c = lax.axis_index("c")
    s = lax.axis_index("s")
    wid = s * NC + c

    pltpu.sync_copy(zrows_hbm, vrows)
    for j in range(NR // CH):
        pltpu.sync_copy(vrows, num_s.at[pl.ds(s * NR + j * CH, CH)])
    pltpu.sync_copy(zvec_hbm, mloc.at[pl.ds(0, NR)])
    pltpu.sync_copy(mloc.at[pl.ds(0, NR)], den_s.at[pl.ds(s * NR, NR)])
    pltpu.sync_copy(m_hbm, mloc)
    plsc.subcore_barrier()

    def chunk_body(ci, carry):
        base = wid * EPW + ci * CH
        pltpu.sync_copy(src_hbm.at[pl.ds(base, CH)], idxs)
        pltpu.sync_copy(dst_hbm.at[pl.ds(base, CH)], idxd)
        pltpu.sync_copy(e_hbm.at[pl.ds(base, CH)], ebuf)
        cpv = pltpu.async_copy(v_hbm.at[idxs], vrows, semv)
        for g in range(GP):
            dstv = idxd[pl.ds(g * L, L)]
            mv = plsc.load_gather(mloc, [dstv])
            ev = ebuf[pl.ds(g * L, L)]
            exbuf[pl.ds(g * L, L)] = jnp.exp(ev - mv)
        cpv.wait()

        def scale_body(r, carry2):
            exr = plsc.load_gather(exbuf, [jnp.full((L,), r, jnp.int32)])
            for cc in range(D // L):
                sl = pl.ds(cc * L, L)
                vrows[r, sl] = vrows[r, sl] * exr
            return carry2
        lax.fori_loop(0, CH, scale_body, 0)
        pltpu.sync_copy(vrows, num_s.at[idxd], add=True)
        pltpu.sync_copy(exbuf, den_s.at[idxd], add=True)
        return carry
    lax.fori_loop(0, NCH, chunk_body, 0)
    plsc.subcore_barrier()
    pltpu.sync_copy(num_s.at[pl.ds(s * NR, NR)],
                    nump_out.at[c, pl.ds(s * NR, NR)])
    pltpu.sync_copy(den_s.at[pl.ds(s * NR, NR)],
                    denp_out.at[pl.ds(c * NP + s * NR, NR)])


def _k3(v, src, dst, e, m, zrows, zvec):
    return pl.kernel(
        _k3_body,
        out_type=(jax.ShapeDtypeStruct((NC, NP, D), jnp.float32),
                  jax.ShapeDtypeStruct((NC * NP,), jnp.float32)),
        mesh=_mesh,
        scratch_types=[
            pltpu.VMEM((NP,), jnp.float32),
            pltpu.VMEM((CH,), jnp.int32),
            pltpu.VMEM((CH,), jnp.int32),
            pltpu.VMEM((CH,), jnp.float32),
            pltpu.VMEM((CH,), jnp.float32),
            pltpu.VMEM((CH, D), jnp.float32),
            pltpu.VMEM_SHARED((NP, D), jnp.float32),
            pltpu.VMEM_SHARED((NP,), jnp.float32),
            pltpu.SemaphoreType.DMA,
        ],
        compiler_params=_sc_params,
    )(v, src, dst, e, m, zrows, zvec)


def _norm_body(num_ref, den_ref, h_ref):
    n = num_ref[0] + num_ref[1]
    d = den_ref[0] + den_ref[1]
    h_ref[...] = n / (d[:, None] + 1e-16)


def _norm(num_p, den_p):
    rb = 1024
    return pl.pallas_call(
        _norm_body,
        grid=(NP // rb,),
        in_specs=[pl.BlockSpec((NC, rb, D), lambda i: (0, i, 0)),
                  pl.BlockSpec((NC, rb), lambda i: (0, i))],
        out_specs=pl.BlockSpec((rb, D), lambda i: (i, 0)),
        out_shape=jax.ShapeDtypeStruct((NP, D), jnp.float32),
    )(num_p, den_p)


def kernel(x, edge_index, Wk, Wq, Wv):
    src = edge_index[0].astype(jnp.int32)
    dst = edge_index[1].astype(jnp.int32)
    k, q, v = _kqv(x, Wk, Wq, Wv)
    e, m_part = _k1(k, q, src, dst)
    m = _k2(m_part)
    zrows = jnp.zeros((CH, D), jnp.float32)
    zvec = jnp.zeros((NR,), jnp.float32)
    num_p, den_p = _k3(v, src, dst, e, m, zrows, zvec)
    h = _norm(num_p, den_p.reshape(NC, NP))
    return h[:N]


# speedup vs baseline: 3.9741x; 1.0021x over previous
import jax
import jax.numpy as jnp
from jax import lax
from jax.experimental import pallas as pl
from jax.experimental.pallas import tpu as pltpu
from jax.experimental.pallas import tpu_sc as plsc

N = 10000
E = 320000
D = 128
NP = 10240
NC = 2
NS = 16
L = 16
NW = NC * NS
EPW = E // NW
CH = 80
NCH = EPW // CH
GP = CH // L
NR = NP // NS
NPW = NP // NW

_mesh = plsc.VectorSubcoreMesh(core_axis_name="c", subcore_axis_name="s")
_sc_params = pltpu.CompilerParams(needs_layout_passes=False)


def _kqv_body(x_ref, wk_ref, wq_ref, wv_ref, k_ref, q_ref, v_ref):
    xb = x_ref[...]
    dn = (((1,), (1,)), ((), ()))
    k_ref[...] = lax.dot_general(xb, wk_ref[...], dn,
                                 preferred_element_type=jnp.float32)
    q_ref[...] = lax.dot_general(xb, wq_ref[...], dn,
                                 preferred_element_type=jnp.float32)
    v_ref[...] = lax.dot_general(xb, wv_ref[...], dn,
                                 preferred_element_type=jnp.float32)


def _kqv(x, wk, wq, wv):
    rb = 2000
    wspec = pl.BlockSpec((D, D), lambda i: (0, 0))
    xspec = pl.BlockSpec((rb, D), lambda i: (i, 0))
    return pl.pallas_call(
        _kqv_body,
        grid=(N // rb,),
        in_specs=[xspec, wspec, wspec, wspec],
        out_specs=[xspec, xspec, xspec],
        out_shape=[jax.ShapeDtypeStruct((N, D), jnp.float32)] * 3,
    )(x, wk, wq, wv)


def _k1_body(k_hbm, q_hbm, src_hbm, dst_hbm, e_out, mpart_out,
             idxs, idxd, krows, qrows, ebuf, mpriv, sem1, sem2):
    c = lax.axis_index("c")
    s = lax.axis_index("s")
    wid = s * NC + c
    lanes = lax.broadcasted_iota(jnp.int32, (L,), 0)

    def init_body(i, carry):
        mpriv[pl.ds(i * L, L)] = jnp.full((L,), -1e30, jnp.float32)
        return carry
    lax.fori_loop(0, NP // L, init_body, 0)

    def chunk_body(ci, carry):
        base = wid * EPW + ci * CH
        pltpu.sync_copy(src_hbm.at[pl.ds(base, CH)], idxs)
        pltpu.sync_copy(dst_hbm.at[pl.ds(base, CH)], idxd)
        cpk = pltpu.async_copy(k_hbm.at[idxs], krows, sem1)
        cpq = pltpu.async_copy(q_hbm.at[idxd], qrows, sem2)
        cpk.wait()
        cpq.wait()
        for g in range(GP):
            rows = lanes + g * L

            def dot_body(j16, acc):
                for dj in range(16):
                    jv = j16 * 16 + jnp.full((L,), dj, jnp.int32)
                    kv = plsc.load_gather(krows, [rows, jv])
                    qv = plsc.load_gather(qrows, [rows, jv])
                    acc = acc + kv * qv
                return acc

            acc = lax.fori_loop(0, D // 16, dot_body,
                                jnp.zeros((L,), jnp.float32))
            ev = jnp.where(acc >= 0, acc, acc * 0.01)
            ebuf[pl.ds(g * L, L)] = ev
            dstv = idxd[pl.ds(g * L, L)]

            def mx_cond(pend):
                return jnp.sum(pend.astype(jnp.int32)) > 0

            def mx_body(pend):
                cur = plsc.load_gather(mpriv, [dstv])
                need = jnp.logical_and(pend, ev > cur)
                plsc.store_scatter(mpriv, [dstv], jnp.maximum(cur, ev),
                                   mask=need)
                cur2 = plsc.load_gather(mpriv, [dstv])
                return cur2 < ev

            lax.while_loop(mx_cond, mx_body, jnp.ones((L,), jnp.bool_))
        pltpu.sync_copy(ebuf, e_out.at[pl.ds(base, CH)])
        return carry
    lax.fori_loop(0, NCH, chunk_body, 0)
    pltpu.sync_copy(mpriv, mpart_out.at[pl.ds(wid * NP, NP)])


def _k1(k, q, src, dst):
    return pl.kernel(
        _k1_body,
        out_type=(jax.ShapeDtypeStruct((E,), jnp.float32),
                  jax.ShapeDtypeStruct((NW * NP,), jnp.float32)),
        mesh=_mesh,
        scratch_types=[
            pltpu.VMEM((CH,), jnp.int32),
            pltpu.VMEM((CH,), jnp.int32),
            pltpu.VMEM((CH, D), jnp.float32),
            pltpu.VMEM((CH, D), jnp.float32),
            pltpu.VMEM((CH,), jnp.float32),
            pltpu.VMEM((NP,), jnp.float32),
            pltpu.SemaphoreType.DMA,
            pltpu.SemaphoreType.DMA,
        ],
        compiler_params=_sc_params,
    )(k, q, src, dst)


def _k2_body(mpart_in, m_out, acc, tmp):
    c = lax.axis_index("c")
    s = lax.axis_index("s")
    wid = s * NC + c
    base = wid * NPW
    pltpu.sync_copy(mpart_in.at[pl.ds(base, NPW)], acc)

    def red(j, carry):
        pltpu.sync_copy(mpart_in.at[pl.ds(j * NP + base, NPW)], tmp)
        for t in range(NPW // L):
            sl = pl.ds(t * L, L)
            acc[sl] = jnp.maximum(acc[sl], tmp[sl])
        return carry
    lax.fori_loop(1, NW, red, 0)
    pltpu.sync_copy(acc, m_out.at[pl.ds(base, NPW)])


def _k2(m_part):
    return pl.kernel(
        _k2_body,
        out_type=jax.ShapeDtypeStruct((NP,), jnp.float32),
        mesh=_mesh,
        scratch_types=[
            pltpu.VMEM((NPW,), jnp.float32),
            pltpu.VMEM((NPW,), jnp.float32),
        ],
        compiler_params=_sc_params,
    )(m_part)


def _k3_body(v_hbm, src_hbm, dst_hbm, e_hbm, m_hbm, zrows_hbm, zvec_hbm,
             nump_out, denp_out,
             mloc, idxs, idxd, ebuf, exbuf, vrows, num_s, den_s, semv):
    c = lax.axis_index("c")
    s = lax.axis_index("s")
    wid = s * NC + c

    pltpu.sync_copy(zrows_hbm, vrows)
    for j in range(NR // CH):
        pltpu.sync_copy(vrows, num_s.at[pl.ds(s * NR + j * CH, CH)])
    pltpu.sync_copy(zvec_hbm, mloc.at[pl.ds(0, NR)])
    pltpu.sync_copy(mloc.at[pl.ds(0, NR)], den_s.at[pl.ds(s * NR, NR)])
    pltpu.sync_copy(m_hbm, mloc)
    plsc.subcore_barrier()

    def chunk_body(ci, carry):
        base = wid * EPW + ci * CH
        pltpu.sync_copy(src_hbm.at[pl.ds(base, CH)], idxs)
        pltpu.sync_copy(dst_hbm.at[pl.ds(base, CH)], idxd)
        pltpu.sync_copy(e_hbm.at[pl.ds(base, CH)], ebuf)
        cpv = pltpu.async_copy(v_hbm.at[idxs], vrows, semv)
        for g in range(GP):
            dstv = idxd[pl.ds(g * L, L)]
            mv = plsc.load_gather(mloc, [dstv])
            ev = ebuf[pl.ds(g * L, L)]
            exbuf[pl.ds(g * L, L)] = jnp.exp(ev - mv)
        cpv.wait()

        def scale_body(r, carry2):
            exr = plsc.load_gather(exbuf, [jnp.full((L,), r, jnp.int32)])
            for cc in range(D // L):
                sl = pl.ds(cc * L, L)
                vrows[r, sl] = vrows[r, sl] * exr
            return carry2
        lax.fori_loop(0, CH, scale_body, 0)
        pltpu.sync_copy(vrows, num_s.at[idxd], add=True)
        pltpu.sync_copy(exbuf, den_s.at[idxd], add=True)
        return carry
    lax.fori_loop(0, NCH, chunk_body, 0)
    plsc.subcore_barrier()
    pltpu.sync_copy(num_s.at[pl.ds(s * NR, NR)],
                    nump_out.at[c, pl.ds(s * NR, NR)])
    pltpu.sync_copy(den_s.at[pl.ds(s * NR, NR)],
                    denp_out.at[pl.ds(c * NP + s * NR, NR)])


def _k3(v, src, dst, e, m, zrows, zvec):
    return pl.kernel(
        _k3_body,
        out_type=(jax.ShapeDtypeStruct((NC, NP, D), jnp.float32),
                  jax.ShapeDtypeStruct((NC * NP,), jnp.float32)),
        mesh=_mesh,
        scratch_types=[
            pltpu.VMEM((NP,), jnp.float32),
            pltpu.VMEM((CH,), jnp.int32),
            pltpu.VMEM((CH,), jnp.int32),
            pltpu.VMEM((CH,), jnp.float32),
            pltpu.VMEM((CH,), jnp.float32),
            pltpu.VMEM((CH, D), jnp.float32),
            pltpu.VMEM_SHARED((NP, D), jnp.float32),
            pltpu.VMEM_SHARED((NP,), jnp.float32),
            pltpu.SemaphoreType.DMA,
        ],
        compiler_params=_sc_params,
    )(v, src, dst, e, m, zrows, zvec)


def _norm_body(num_ref, den_ref, h_ref):
    n = num_ref[0] + num_ref[1]
    d = den_ref[0] + den_ref[1]
    h_ref[...] = n / (d[:, None] + 1e-16)


def _norm(num_p, den_p):
    rb = 1024
    return pl.pallas_call(
        _norm_body,
        grid=(NP // rb,),
        in_specs=[pl.BlockSpec((NC, rb, D), lambda i: (0, i, 0)),
                  pl.BlockSpec((NC, rb), lambda i: (0, i))],
        out_specs=pl.BlockSpec((rb, D), lambda i: (i, 0)),
        out_shape=jax.ShapeDtypeStruct((NP, D), jnp.float32),
    )(num_p, den_p)


def kernel(x, edge_index, Wk, Wq, Wv):
    src = edge_index[0].astype(jnp.int32)
    dst = edge_index[1].astype(jnp.int32)
    k, q, v = _kqv(x, Wk, Wq, Wv)
    e, m_part = _k1(k, q, src, dst)
    m = _k2(m_part)
    zrows = jnp.zeros((CH, D), jnp.float32)
    zvec = jnp.zeros((NR,), jnp.float32)
    num_p, den_p = _k3(v, src, dst, e, m, zrows, zvec)
    h = _norm(num_p, den_p.reshape(NC, NP))
    return h[:N]


# speedup vs baseline: 5.3994x; 1.3587x over previous
import jax
import jax.numpy as jnp
from jax import lax
from jax.experimental import pallas as pl
from jax.experimental.pallas import tpu as pltpu
from jax.experimental.pallas import tpu_sc as plsc

N = 10000
E = 320000
D = 128
NP = 10240
NC = 2
NS = 16
L = 16
NW = NC * NS
EPW = E // NW
CH = 80
NCH = EPW // CH
GP = CH // L
NR = NP // NS
NPW = NP // NW
NVP = 2 * NP
NWP = 256

_mesh = plsc.VectorSubcoreMesh(core_axis_name="c", subcore_axis_name="s")
_sc_params = pltpu.CompilerParams(
    needs_layout_passes=False,
    allow_input_fusion=(False,) * 8,
)


def _kqv_body(x_ref, wk_ref, wq_ref, wv_ref, k_ref, q_ref, v_ref):
    xb = x_ref[...]
    dn = (((1,), (1,)), ((), ()))
    k_ref[...] = lax.dot_general(xb, wk_ref[...], dn,
                                 preferred_element_type=jnp.float32)
    q_ref[...] = lax.dot_general(xb, wq_ref[...], dn,
                                 preferred_element_type=jnp.float32)
    v_ref[...] = lax.dot_general(xb, wv_ref[...], dn,
                                 preferred_element_type=jnp.float32)


def _kqv(x, wk, wq, wv):
    rb = 2000
    wspec = pl.BlockSpec((D, D), lambda i: (0, 0))
    xspec = pl.BlockSpec((rb, D), lambda i: (i, 0))
    return pl.pallas_call(
        _kqv_body,
        grid=(N // rb,),
        in_specs=[xspec, wspec, wspec, wspec],
        out_specs=[xspec, xspec, xspec],
        out_shape=[jax.ShapeDtypeStruct((N, D), jnp.float32)] * 3,
    )(x, wk, wq, wv)


def _copy_idx(src1d, base, dst_small):
    for g in range(GP):
        sl = pl.ds(g * L, L)
        dst_small[sl] = src1d[pl.ds(base + g * L, L)]


def _k1_body(k_hbm, q_hbm, src_hbm, dst_hbm, e_out, mpart_out,
             idxs1, idxd1, mpriv, kra, qra, krb, qrb,
             isa, ida, isb, idb, eba, ebb,
             semka, semqa, semkb, semqb, semea, semeb):
    c = lax.axis_index("c")
    s = lax.axis_index("s")
    wid = s * NC + c
    lanes = lax.broadcasted_iota(jnp.int32, (L,), 0)

    def init_body(i, carry):
        mpriv[pl.ds(i * L, L)] = jnp.full((L,), -1e30, jnp.float32)
        return carry
    lax.fori_loop(0, NP // L, init_body, 0)

    pltpu.sync_copy(src_hbm.at[pl.ds(wid * EPW, EPW)], idxs1)
    pltpu.sync_copy(dst_hbm.at[pl.ds(wid * EPW, EPW)], idxd1)

    def compute_chunk(ci, krows, qrows, ebuf, seme):
        @pl.when(ci >= 2)
        def _():
            pltpu.make_async_copy(ebuf, e_out.at[pl.ds(0, CH)], seme).wait()
        for g in range(GP):
            rows = lanes + g * L

            def dot_body(j16, acc):
                for dj in range(16):
                    jv = j16 * 16 + jnp.full((L,), dj, jnp.int32)
                    kv = plsc.load_gather(krows, [rows, jv])
                    qv = plsc.load_gather(qrows, [rows, jv])
                    acc = acc + kv * qv
                return acc

            acc = lax.fori_loop(0, D // 16, dot_body,
                                jnp.zeros((L,), jnp.float32))
            ev = jnp.where(acc >= 0, acc, acc * 0.01)
            ebuf[pl.ds(g * L, L)] = ev
            dstv = idxd1[pl.ds(ci * CH + g * L, L)]

            def mx_cond(pend):
                return jnp.sum(pend.astype(jnp.int32)) > 0

            def mx_body(pend):
                cur = plsc.load_gather(mpriv, [dstv])
                need = jnp.logical_and(pend, ev > cur)
                plsc.store_scatter(mpriv, [dstv], jnp.maximum(cur, ev),
                                   mask=need)
                cur2 = plsc.load_gather(mpriv, [dstv])
                return cur2 < ev

            lax.while_loop(mx_cond, mx_body, jnp.ones((L,), jnp.bool_))
        pltpu.async_copy(ebuf, e_out.at[pl.ds(wid * EPW + ci * CH, CH)],
                         seme)

    def issue_a(ci):
        _copy_idx(idxs1, ci * CH, isa)
        _copy_idx(idxd1, ci * CH, ida)
        pltpu.async_copy(k_hbm.at[isa], kra, semka)
        pltpu.async_copy(q_hbm.at[ida], qra, semqa)

    def wait_a():
        pltpu.make_async_copy(k_hbm.at[isa], kra, semka).wait()
        pltpu.make_async_copy(q_hbm.at[ida], qra, semqa).wait()

    issue_a(0)

    def body(i, carry):
        c0 = 2 * i
        _copy_idx(idxs1, (c0 + 1) * CH, isb)
        _copy_idx(idxd1, (c0 + 1) * CH, idb)
        cpk = pltpu.async_copy(k_hbm.at[isb], krb, semkb)
        cpq = pltpu.async_copy(q_hbm.at[idb], qrb, semqb)
        wait_a()
        compute_chunk(c0, kra, qra, eba, semea)
        issue_a(c0 + 2)
        cpk.wait()
        cpq.wait()
        compute_chunk(c0 + 1, krb, qrb, ebb, semeb)
        return carry
    lax.fori_loop(0, (NCH - 1) // 2, body, 0)
    wait_a()
    compute_chunk(NCH - 1, kra, qra, eba, semea)
    pltpu.make_async_copy(eba, e_out.at[pl.ds(0, CH)], semea).wait()
    pltpu.make_async_copy(ebb, e_out.at[pl.ds(0, CH)], semeb).wait()
    pltpu.sync_copy(mpriv, mpart_out.at[pl.ds(wid * NP, NP)])


def _k1(k, q, src1, dst1):
    return pl.kernel(
        _k1_body,
        out_type=(jax.ShapeDtypeStruct((E,), jnp.float32),
                  jax.ShapeDtypeStruct((NW * NP,), jnp.float32)),
        mesh=_mesh,
        scratch_types=[
            pltpu.VMEM((EPW,), jnp.int32),
            pltpu.VMEM((EPW,), jnp.int32),
            pltpu.VMEM((NP,), jnp.float32),
            pltpu.VMEM((CH, D), jnp.float32),
            pltpu.VMEM((CH, D), jnp.float32),
            pltpu.VMEM((CH, D), jnp.float32),
            pltpu.VMEM((CH, D), jnp.float32),
            pltpu.VMEM((CH,), jnp.int32),
            pltpu.VMEM((CH,), jnp.int32),
            pltpu.VMEM((CH,), jnp.int32),
            pltpu.VMEM((CH,), jnp.int32),
            pltpu.VMEM((CH,), jnp.float32),
            pltpu.VMEM((CH,), jnp.float32),
            pltpu.SemaphoreType.DMA,
            pltpu.SemaphoreType.DMA,
            pltpu.SemaphoreType.DMA,
            pltpu.SemaphoreType.DMA,
            pltpu.SemaphoreType.DMA,
            pltpu.SemaphoreType.DMA,
        ],
        compiler_params=_sc_params,
    )(k, q, src1, dst1)


def _k2_body(mpart_in, m_out, acc, tmp):
    c = lax.axis_index("c")
    s = lax.axis_index("s")
    wid = s * NC + c
    base = wid * NPW
    pltpu.sync_copy(mpart_in.at[pl.ds(base, NPW)], acc)

    def red(j, carry):
        pltpu.sync_copy(mpart_in.at[pl.ds(j * NP + base, NPW)], tmp)
        for t in range(NPW // L):
            sl = pl.ds(t * L, L)
            acc[sl] = jnp.maximum(acc[sl], tmp[sl])
        return carry
    lax.fori_loop(1, NW, red, 0)
    pltpu.sync_copy(acc, m_out.at[pl.ds(base, NPW)])


def _tc_relay_body(x_ref, y_ref):
    y_ref[...] = x_ref[...] * 1.0


def _tc_relay(x):
    return pl.pallas_call(
        _tc_relay_body,
        out_shape=jax.ShapeDtypeStruct((NP,), jnp.float32),
    )(x)


def _k2(m_part):
    return pl.kernel(
        _k2_body,
        out_type=jax.ShapeDtypeStruct((NP,), jnp.float32),
        mesh=_mesh,
        scratch_types=[
            pltpu.VMEM((NPW,), jnp.float32),
            pltpu.VMEM((NPW,), jnp.float32),
        ],
        compiler_params=_sc_params,
    )(m_part)


def _k3_body(v_hbm, src_hbm, dst_hbm, e_hbm, m_hbm, zrows_hbm, zvec_hbm,
             nump_out, denp_out,
             mloc, idxs1, exa, exb, vra, vrb, isa, isb,
             idda, iddb, eba, ebb, num_s, den_s, semva, semvb):
    c = lax.axis_index("c")
    s = lax.axis_index("s")
    wid = s * NC + c

    pltpu.sync_copy(zrows_hbm, vra)
    for j in range(NR // CH):
        pltpu.sync_copy(vra, num_s.at[pl.ds(s * NR + j * CH, CH)])
    pltpu.sync_copy(zvec_hbm, mloc.at[pl.ds(0, NR)])
    pltpu.sync_copy(mloc.at[pl.ds(0, NR)], den_s.at[pl.ds(s * NR, NR)])
    pltpu.sync_copy(src_hbm.at[pl.ds(wid * EPW, EPW)], idxs1)
    pltpu.sync_copy(m_hbm, mloc)
    plsc.subcore_barrier()

    def compute_chunk(ci, vrows, exbuf, idd, ebuf):
        for g in range(GP):
            dstv = idd[pl.ds(g * L, L)]
            mv = plsc.load_gather(mloc, [dstv])
            ev = ebuf[pl.ds(g * L, L)]
            exbuf[pl.ds(g * L, L)] = jnp.exp(ev - mv)

        def scale_body(r, carry2):
            exr = plsc.load_gather(exbuf, [jnp.full((L,), r, jnp.int32)])
            for cc in range(D // L):
                sl = pl.ds(cc * L, L)
                vrows[r, sl] = vrows[r, sl] * exr
            return carry2
        lax.fori_loop(0, CH, scale_body, 0)
        pltpu.sync_copy(vrows, num_s.at[idd], add=True)
        pltpu.sync_copy(exbuf, den_s.at[idd], add=True)

    def issue_a(ci):
        _copy_idx(idxs1, ci * CH, isa)
        base = wid * EPW + ci * CH
        pltpu.async_copy(v_hbm.at[isa], vra, semva)
        pltpu.async_copy(dst_hbm.at[pl.ds(base, CH)], idda, semva)
        pltpu.async_copy(e_hbm.at[pl.ds(base, CH)], eba, semva)

    def wait_a():
        pltpu.make_async_copy(v_hbm.at[isa], vra, semva).wait()
        pltpu.make_async_copy(dst_hbm.at[pl.ds(0, CH)], idda, semva).wait()
        pltpu.make_async_copy(e_hbm.at[pl.ds(0, CH)], eba, semva).wait()

    issue_a(0)

    def body(i, carry):
        c0 = 2 * i
        _copy_idx(idxs1, (c0 + 1) * CH, isb)
        baseb = wid * EPW + (c0 + 1) * CH
        cpv = pltpu.async_copy(v_hbm.at[isb], vrb, semvb)
        cpd = pltpu.async_copy(dst_hbm.at[pl.ds(baseb, CH)], iddb, semvb)
        cpe = pltpu.async_copy(e_hbm.at[pl.ds(baseb, CH)], ebb, semvb)
        wait_a()
        compute_chunk(c0, vra, exa, idda, eba)
        issue_a(c0 + 2)
        cpv.wait()
        cpd.wait()
        cpe.wait()
        compute_chunk(c0 + 1, vrb, exb, iddb, ebb)
        return carry
    lax.fori_loop(0, (NCH - 1) // 2, body, 0)
    wait_a()
    compute_chunk(NCH - 1, vra, exa, idda, eba)
    plsc.subcore_barrier()
    pltpu.sync_copy(num_s.at[pl.ds(s * NR, NR)],
                    nump_out.at[c, pl.ds(s * NR, NR)])
    pltpu.sync_copy(den_s.at[pl.ds(s * NR, NR)],
                    denp_out.at[pl.ds(c * NP + s * NR, NR)])


def _k3(v, src1, dst1, e1, m, zrows, zvec):
    return pl.kernel(
        _k3_body,
        out_type=(jax.ShapeDtypeStruct((NC, NP, D), jnp.float32),
                  jax.ShapeDtypeStruct((NC * NP,), jnp.float32)),
        mesh=_mesh,
        scratch_types=[
            pltpu.VMEM((NP,), jnp.float32),
            pltpu.VMEM((EPW,), jnp.int32),
            pltpu.VMEM((CH,), jnp.float32),
            pltpu.VMEM((CH,), jnp.float32),
            pltpu.VMEM((CH, D), jnp.float32),
            pltpu.VMEM((CH, D), jnp.float32),
            pltpu.VMEM((CH,), jnp.int32),
            pltpu.VMEM((CH,), jnp.int32),
            pltpu.VMEM((CH,), jnp.int32),
            pltpu.VMEM((CH,), jnp.int32),
            pltpu.VMEM((CH,), jnp.float32),
            pltpu.VMEM((CH,), jnp.float32),
            pltpu.VMEM_SHARED((NP, D), jnp.float32),
            pltpu.VMEM_SHARED((NP,), jnp.float32),
            pltpu.SemaphoreType.DMA,
            pltpu.SemaphoreType.DMA,
        ],
        compiler_params=_sc_params,
    )(v, src1, dst1, e1, m, zrows, zvec)


def _norm_body(num_ref, den_ref, h_ref):
    n = num_ref[0] + num_ref[1]
    d = den_ref[0] + den_ref[1]
    h_ref[...] = n / (d[:, None] + 1e-16)


def _norm(num_p, den_p):
    rb = 1024
    return pl.pallas_call(
        _norm_body,
        grid=(pl.cdiv(N, rb),),
        in_specs=[pl.BlockSpec((NC, rb, D), lambda i: (0, i, 0)),
                  pl.BlockSpec((NC, rb), lambda i: (0, i))],
        out_specs=pl.BlockSpec((rb, D), lambda i: (i, 0)),
        out_shape=jax.ShapeDtypeStruct((N, D), jnp.float32),
    )(num_p, den_p)


def kernel(x, edge_index, Wk, Wq, Wv):
    src1 = edge_index[0].astype(jnp.int32)
    dst1 = edge_index[1].astype(jnp.int32)
    k, q, v = _kqv(x, Wk, Wq, Wv)
    e1, m_part = _k1(k, q, src1, dst1)
    m = _tc_relay(_k2(m_part))
    zrows = jnp.zeros((CH, D), jnp.float32)
    zvec = jnp.zeros((NR,), jnp.float32)
    num_p, den_p = _k3(v, src1, dst1, e1, m, zrows, zvec)
    return _norm(num_p, den_p.reshape(NC, NP))


# speedup vs baseline: 15.6358x; 2.8958x over previous
import jax
import jax.numpy as jnp
from jax import lax
from jax.experimental import pallas as pl
from jax.experimental.pallas import tpu as pltpu
from jax.experimental.pallas import tpu_sc as plsc

N = 10000
E = 320000
D = 128
NP = 10240
NC = 2
NS = 16
L = 16
NW = NC * NS
EPW = E // NW
CH = 80
NCH = EPW // CH
GP = CH // L
NR = NP // NS
NPW = NP // NW
NVP = 2 * NP
NWP = 256

_mesh = plsc.VectorSubcoreMesh(core_axis_name="c", subcore_axis_name="s")
_sc_params = pltpu.CompilerParams(
    needs_layout_passes=False,
    allow_input_fusion=(False,) * 8,
)


def _kqv_body(x_ref, wk_ref, wq_ref, wv_ref, k_ref, q_ref, v_ref):
    xb = x_ref[...]
    dn = (((1,), (1,)), ((), ()))
    k_ref[...] = lax.dot_general(xb, wk_ref[...], dn,
                                 preferred_element_type=jnp.float32)
    q_ref[...] = lax.dot_general(xb, wq_ref[...], dn,
                                 preferred_element_type=jnp.float32)
    v_ref[...] = lax.dot_general(xb, wv_ref[...], dn,
                                 preferred_element_type=jnp.float32)


def _kqv(x, wk, wq, wv):
    rb = 2000
    wspec = pl.BlockSpec((D, D), lambda i: (0, 0))
    xspec = pl.BlockSpec((rb, D), lambda i: (i, 0))
    return pl.pallas_call(
        _kqv_body,
        grid=(N // rb,),
        in_specs=[xspec, wspec, wspec, wspec],
        out_specs=[xspec, xspec, xspec],
        out_shape=[jax.ShapeDtypeStruct((N, D), jnp.float32)] * 3,
    )(x, wk, wq, wv)


def _copy_idx(src1d, base, dst_small):
    for g in range(GP):
        sl = pl.ds(g * L, L)
        dst_small[sl] = src1d[pl.ds(base + g * L, L)]


def _k1_body(k_hbm, q_hbm, src_hbm, dst_hbm, e_out, mpart_out,
             idxs1, idxd1, mpriv, kra, qra, krb, qrb,
             isa, ida, isb, idb, eba, ebb,
             semka, semqa, semkb, semqb, semea, semeb):
    c = lax.axis_index("c")
    s = lax.axis_index("s")
    wid = s * NC + c
    lanes = lax.broadcasted_iota(jnp.int32, (L,), 0)

    def init_body(i, carry):
        mpriv[pl.ds(i * L, L)] = jnp.full((L,), -1e30, jnp.float32)
        return carry
    lax.fori_loop(0, NP // L, init_body, 0)

    pltpu.sync_copy(src_hbm.at[pl.ds(wid * EPW, EPW)], idxs1)
    pltpu.sync_copy(dst_hbm.at[pl.ds(wid * EPW, EPW)], idxd1)

    def compute_chunk(ci, krows, qrows, ebuf, seme):
        @pl.when(ci >= 2)
        def _():
            pltpu.make_async_copy(ebuf, e_out.at[pl.ds(0, CH)], seme).wait()

        def group_body(g, carry):
            e16 = jnp.zeros((L,), jnp.float32)
            for rr in range(L):
                r = g * L + rr
                acc = krows[r, pl.ds(0, L)] * qrows[r, pl.ds(0, L)]
                for cc in range(1, D // L):
                    sl = pl.ds(cc * L, L)
                    acc = acc + krows[r, sl] * qrows[r, sl]
                er = jnp.sum(acc)
                er = jnp.where(er >= 0, er, er * 0.01)
                e16 = jnp.where(lanes == rr, er, e16)
            ebuf[pl.ds(g * L, L)] = e16
            ev = e16
            dstv = idxd1[pl.ds(ci * CH + g * L, L)]

            def mx_cond(pend):
                return jnp.sum(pend.astype(jnp.int32)) > 0

            def mx_body(pend):
                cur = plsc.load_gather(mpriv, [dstv])
                need = jnp.logical_and(pend, ev > cur)
                plsc.store_scatter(mpriv, [dstv], jnp.maximum(cur, ev),
                                   mask=need)
                cur2 = plsc.load_gather(mpriv, [dstv])
                return cur2 < ev

            lax.while_loop(mx_cond, mx_body, jnp.ones((L,), jnp.bool_))
            return carry
        lax.fori_loop(0, GP, group_body, 0)
        pltpu.async_copy(ebuf, e_out.at[pl.ds(wid * EPW + ci * CH, CH)],
                         seme)

    def issue_a(ci):
        _copy_idx(idxs1, ci * CH, isa)
        _copy_idx(idxd1, ci * CH, ida)
        pltpu.async_copy(k_hbm.at[isa], kra, semka)
        pltpu.async_copy(q_hbm.at[ida], qra, semqa)

    def wait_a():
        pltpu.make_async_copy(k_hbm.at[isa], kra, semka).wait()
        pltpu.make_async_copy(q_hbm.at[ida], qra, semqa).wait()

    issue_a(0)

    def body(i, carry):
        c0 = 2 * i
        _copy_idx(idxs1, (c0 + 1) * CH, isb)
        _copy_idx(idxd1, (c0 + 1) * CH, idb)
        cpk = pltpu.async_copy(k_hbm.at[isb], krb, semkb)
        cpq = pltpu.async_copy(q_hbm.at[idb], qrb, semqb)
        wait_a()
        compute_chunk(c0, kra, qra, eba, semea)
        issue_a(c0 + 2)
        cpk.wait()
        cpq.wait()
        compute_chunk(c0 + 1, krb, qrb, ebb, semeb)
        return carry
    lax.fori_loop(0, (NCH - 1) // 2, body, 0)
    wait_a()
    compute_chunk(NCH - 1, kra, qra, eba, semea)
    pltpu.make_async_copy(eba, e_out.at[pl.ds(0, CH)], semea).wait()
    pltpu.make_async_copy(ebb, e_out.at[pl.ds(0, CH)], semeb).wait()
    pltpu.sync_copy(mpriv, mpart_out.at[pl.ds(wid * NP, NP)])


def _k1(k, q, src1, dst1):
    return pl.kernel(
        _k1_body,
        out_type=(jax.ShapeDtypeStruct((E,), jnp.float32),
                  jax.ShapeDtypeStruct((NW * NP,), jnp.float32)),
        mesh=_mesh,
        scratch_types=[
            pltpu.VMEM((EPW,), jnp.int32),
            pltpu.VMEM((EPW,), jnp.int32),
            pltpu.VMEM((NP,), jnp.float32),
            pltpu.VMEM((CH, D), jnp.float32),
            pltpu.VMEM((CH, D), jnp.float32),
            pltpu.VMEM((CH, D), jnp.float32),
            pltpu.VMEM((CH, D), jnp.float32),
            pltpu.VMEM((CH,), jnp.int32),
            pltpu.VMEM((CH,), jnp.int32),
            pltpu.VMEM((CH,), jnp.int32),
            pltpu.VMEM((CH,), jnp.int32),
            pltpu.VMEM((CH,), jnp.float32),
            pltpu.VMEM((CH,), jnp.float32),
            pltpu.SemaphoreType.DMA,
            pltpu.SemaphoreType.DMA,
            pltpu.SemaphoreType.DMA,
            pltpu.SemaphoreType.DMA,
            pltpu.SemaphoreType.DMA,
            pltpu.SemaphoreType.DMA,
        ],
        compiler_params=_sc_params,
    )(k, q, src1, dst1)


def _k2_body(mpart_in, m_out, acc, tmp):
    c = lax.axis_index("c")
    s = lax.axis_index("s")
    wid = s * NC + c
    base = wid * NPW
    pltpu.sync_copy(mpart_in.at[pl.ds(base, NPW)], acc)

    def red(j, carry):
        pltpu.sync_copy(mpart_in.at[pl.ds(j * NP + base, NPW)], tmp)
        for t in range(NPW // L):
            sl = pl.ds(t * L, L)
            acc[sl] = jnp.maximum(acc[sl], tmp[sl])
        return carry
    lax.fori_loop(1, NW, red, 0)
    pltpu.sync_copy(acc, m_out.at[pl.ds(base, NPW)])


def _tc_relay_body(x_ref, y_ref):
    y_ref[...] = x_ref[...] * 1.0


def _tc_relay(x):
    return pl.pallas_call(
        _tc_relay_body,
        out_shape=jax.ShapeDtypeStruct((NP,), jnp.float32),
    )(x)


def _k2(m_part):
    return pl.kernel(
        _k2_body,
        out_type=jax.ShapeDtypeStruct((NP,), jnp.float32),
        mesh=_mesh,
        scratch_types=[
            pltpu.VMEM((NPW,), jnp.float32),
            pltpu.VMEM((NPW,), jnp.float32),
        ],
        compiler_params=_sc_params,
    )(m_part)


def _k3_body(v_hbm, src_hbm, dst_hbm, e_hbm, m_hbm, zrows_hbm, zvec_hbm,
             nump_out, denp_out,
             mloc, idxs1, exa, exb, vra, vrb, isa, isb,
             idda, iddb, eba, ebb, num_s, den_s, semva, semvb):
    c = lax.axis_index("c")
    s = lax.axis_index("s")
    wid = s * NC + c

    pltpu.sync_copy(zrows_hbm, vra)
    for j in range(NR // CH):
        pltpu.sync_copy(vra, num_s.at[pl.ds(s * NR + j * CH, CH)])
    pltpu.sync_copy(zvec_hbm, mloc.at[pl.ds(0, NR)])
    pltpu.sync_copy(mloc.at[pl.ds(0, NR)], den_s.at[pl.ds(s * NR, NR)])
    pltpu.sync_copy(src_hbm.at[pl.ds(wid * EPW, EPW)], idxs1)
    pltpu.sync_copy(m_hbm, mloc)
    plsc.subcore_barrier()

    def compute_chunk(ci, vrows, exbuf, idd, ebuf):
        for g in range(GP):
            dstv = idd[pl.ds(g * L, L)]
            mv = plsc.load_gather(mloc, [dstv])
            ev = ebuf[pl.ds(g * L, L)]
            exbuf[pl.ds(g * L, L)] = jnp.exp(ev - mv)

        def scale_body(r, carry2):
            exr = plsc.load_gather(exbuf, [jnp.full((L,), r, jnp.int32)])
            for cc in range(D // L):
                sl = pl.ds(cc * L, L)
                vrows[r, sl] = vrows[r, sl] * exr
            return carry2
        lax.fori_loop(0, CH, scale_body, 0)
        pltpu.sync_copy(vrows, num_s.at[idd], add=True)
        pltpu.sync_copy(exbuf, den_s.at[idd], add=True)

    def issue_a(ci):
        _copy_idx(idxs1, ci * CH, isa)
        base = wid * EPW + ci * CH
        pltpu.async_copy(v_hbm.at[isa], vra, semva)
        pltpu.async_copy(dst_hbm.at[pl.ds(base, CH)], idda, semva)
        pltpu.async_copy(e_hbm.at[pl.ds(base, CH)], eba, semva)

    def wait_a():
        pltpu.make_async_copy(v_hbm.at[isa], vra, semva).wait()
        pltpu.make_async_copy(dst_hbm.at[pl.ds(0, CH)], idda, semva).wait()
        pltpu.make_async_copy(e_hbm.at[pl.ds(0, CH)], eba, semva).wait()

    issue_a(0)

    def body(i, carry):
        c0 = 2 * i
        _copy_idx(idxs1, (c0 + 1) * CH, isb)
        baseb = wid * EPW + (c0 + 1) * CH
        cpv = pltpu.async_copy(v_hbm.at[isb], vrb, semvb)
        cpd = pltpu.async_copy(dst_hbm.at[pl.ds(baseb, CH)], iddb, semvb)
        cpe = pltpu.async_copy(e_hbm.at[pl.ds(baseb, CH)], ebb, semvb)
        wait_a()
        compute_chunk(c0, vra, exa, idda, eba)
        issue_a(c0 + 2)
        cpv.wait()
        cpd.wait()
        cpe.wait()
        compute_chunk(c0 + 1, vrb, exb, iddb, ebb)
        return carry
    lax.fori_loop(0, (NCH - 1) // 2, body, 0)
    wait_a()
    compute_chunk(NCH - 1, vra, exa, idda, eba)
    plsc.subcore_barrier()
    pltpu.sync_copy(num_s.at[pl.ds(s * NR, NR)],
                    nump_out.at[c, pl.ds(s * NR, NR)])
    pltpu.sync_copy(den_s.at[pl.ds(s * NR, NR)],
                    denp_out.at[pl.ds(c * NP + s * NR, NR)])


def _k3(v, src1, dst1, e1, m, zrows, zvec):
    return pl.kernel(
        _k3_body,
        out_type=(jax.ShapeDtypeStruct((NC, NP, D), jnp.float32),
                  jax.ShapeDtypeStruct((NC * NP,), jnp.float32)),
        mesh=_mesh,
        scratch_types=[
            pltpu.VMEM((NP,), jnp.float32),
            pltpu.VMEM((EPW,), jnp.int32),
            pltpu.VMEM((CH,), jnp.float32),
            pltpu.VMEM((CH,), jnp.float32),
            pltpu.VMEM((CH, D), jnp.float32),
            pltpu.VMEM((CH, D), jnp.float32),
            pltpu.VMEM((CH,), jnp.int32),
            pltpu.VMEM((CH,), jnp.int32),
            pltpu.VMEM((CH,), jnp.int32),
            pltpu.VMEM((CH,), jnp.int32),
            pltpu.VMEM((CH,), jnp.float32),
            pltpu.VMEM((CH,), jnp.float32),
            pltpu.VMEM_SHARED((NP, D), jnp.float32),
            pltpu.VMEM_SHARED((NP,), jnp.float32),
            pltpu.SemaphoreType.DMA,
            pltpu.SemaphoreType.DMA,
        ],
        compiler_params=_sc_params,
    )(v, src1, dst1, e1, m, zrows, zvec)


def _norm_body(num_ref, den_ref, h_ref):
    n = num_ref[0] + num_ref[1]
    d = den_ref[0] + den_ref[1]
    h_ref[...] = n / (d[:, None] + 1e-16)


def _norm(num_p, den_p):
    rb = 1024
    return pl.pallas_call(
        _norm_body,
        grid=(pl.cdiv(N, rb),),
        in_specs=[pl.BlockSpec((NC, rb, D), lambda i: (0, i, 0)),
                  pl.BlockSpec((NC, rb), lambda i: (0, i))],
        out_specs=pl.BlockSpec((rb, D), lambda i: (i, 0)),
        out_shape=jax.ShapeDtypeStruct((N, D), jnp.float32),
    )(num_p, den_p)


def kernel(x, edge_index, Wk, Wq, Wv):
    src1 = edge_index[0].astype(jnp.int32)
    dst1 = edge_index[1].astype(jnp.int32)
    k, q, v = _kqv(x, Wk, Wq, Wv)
    e1, m_part = _k1(k, q, src1, dst1)
    m = _tc_relay(_k2(m_part))
    zrows = jnp.zeros((CH, D), jnp.float32)
    zvec = jnp.zeros((NR,), jnp.float32)
    num_p, den_p = _k3(v, src1, dst1, e1, m, zrows, zvec)
    return _norm(num_p, den_p.reshape(NC, NP))


# speedup vs baseline: 17.0163x; 1.0883x over previous
import jax
import jax.numpy as jnp
from jax import lax
from jax.experimental import pallas as pl
from jax.experimental.pallas import tpu as pltpu
from jax.experimental.pallas import tpu_sc as plsc

N = 10000
E = 320000
D = 128
NP = 10240
NC = 2
NS = 16
L = 16
NW = NC * NS
EPW = E // NW
CH = 80
NCH = EPW // CH
GP = CH // L
NR = NP // NS
NPW = NP // NW
NVP = 2 * NP
NWP = 256

_mesh = plsc.VectorSubcoreMesh(core_axis_name="c", subcore_axis_name="s")
_sc_params = pltpu.CompilerParams(
    needs_layout_passes=False,
    allow_input_fusion=(False,) * 8,
)


def _kqv_body(x_ref, wk_ref, wq_ref, wv_ref, k_ref, q_ref, v_ref):
    xb = x_ref[...]
    dn = (((1,), (1,)), ((), ()))
    k_ref[...] = lax.dot_general(xb, wk_ref[...], dn,
                                 preferred_element_type=jnp.float32)
    q_ref[...] = lax.dot_general(xb, wq_ref[...], dn,
                                 preferred_element_type=jnp.float32)
    v_ref[...] = lax.dot_general(xb, wv_ref[...], dn,
                                 preferred_element_type=jnp.float32)


def _kqv(x, wk, wq, wv):
    rb = 2000
    wspec = pl.BlockSpec((D, D), lambda i: (0, 0))
    xspec = pl.BlockSpec((rb, D), lambda i: (i, 0))
    return pl.pallas_call(
        _kqv_body,
        grid=(N // rb,),
        in_specs=[xspec, wspec, wspec, wspec],
        out_specs=[xspec, xspec, xspec],
        out_shape=[jax.ShapeDtypeStruct((N, D), jnp.float32)] * 3,
    )(x, wk, wq, wv)


def _copy_idx(src1d, base, dst_small):
    for g in range(GP):
        sl = pl.ds(g * L, L)
        dst_small[sl] = src1d[pl.ds(base + g * L, L)]


def _k1_body(k_hbm, q_hbm, src_hbm, dst_hbm, e_out, mpart_out,
             idxs1, idxd1, mpriv, kra, qra, krb, qrb,
             isa, ida, isb, idb, eba, ebb,
             semka, semqa, semkb, semqb, semea, semeb):
    c = lax.axis_index("c")
    s = lax.axis_index("s")
    wid = s * NC + c
    lanes = lax.broadcasted_iota(jnp.int32, (L,), 0)

    def init_body(i, carry):
        mpriv[pl.ds(i * L, L)] = jnp.full((L,), -1e30, jnp.float32)
        return carry
    lax.fori_loop(0, NP // L, init_body, 0)

    pltpu.sync_copy(src_hbm.at[pl.ds(wid * EPW, EPW)], idxs1)
    pltpu.sync_copy(dst_hbm.at[pl.ds(wid * EPW, EPW)], idxd1)

    def compute_chunk(ci, krows, qrows, ebuf, seme):
        @pl.when(ci >= 2)
        def _():
            pltpu.make_async_copy(ebuf, e_out.at[pl.ds(0, CH)], seme).wait()

        def group_body(g, carry):
            e16 = jnp.zeros((L,), jnp.float32)
            for rr in range(L):
                r = g * L + rr
                acc = krows[r, pl.ds(0, L)] * qrows[r, pl.ds(0, L)]
                for cc in range(1, D // L):
                    sl = pl.ds(cc * L, L)
                    acc = acc + krows[r, sl] * qrows[r, sl]
                er = jnp.sum(acc)
                er = jnp.where(er >= 0, er, er * 0.01)
                e16 = jnp.where(lanes == rr, er, e16)
            ebuf[pl.ds(g * L, L)] = e16
            ev = e16
            dstv = idxd1[pl.ds(ci * CH + g * L, L)]

            def mx_cond(pend):
                return jnp.sum(pend.astype(jnp.int32)) > 0

            def mx_body(pend):
                cur = plsc.load_gather(mpriv, [dstv])
                need = jnp.logical_and(pend, ev > cur)
                plsc.store_scatter(mpriv, [dstv], jnp.maximum(cur, ev),
                                   mask=need)
                cur2 = plsc.load_gather(mpriv, [dstv])
                return cur2 < ev

            lax.while_loop(mx_cond, mx_body, jnp.ones((L,), jnp.bool_))
            return carry
        lax.fori_loop(0, GP, group_body, 0)
        pltpu.async_copy(ebuf, e_out.at[pl.ds(wid * EPW + ci * CH, CH)],
                         seme)

    def issue_a(ci):
        _copy_idx(idxs1, ci * CH, isa)
        _copy_idx(idxd1, ci * CH, ida)
        pltpu.async_copy(k_hbm.at[isa], kra, semka)
        pltpu.async_copy(q_hbm.at[ida], qra, semqa)

    def wait_a():
        pltpu.make_async_copy(k_hbm.at[isa], kra, semka).wait()
        pltpu.make_async_copy(q_hbm.at[ida], qra, semqa).wait()

    issue_a(0)

    def body(i, carry):
        c0 = 2 * i
        _copy_idx(idxs1, (c0 + 1) * CH, isb)
        _copy_idx(idxd1, (c0 + 1) * CH, idb)
        cpk = pltpu.async_copy(k_hbm.at[isb], krb, semkb)
        cpq = pltpu.async_copy(q_hbm.at[idb], qrb, semqb)
        wait_a()
        compute_chunk(c0, kra, qra, eba, semea)
        issue_a(c0 + 2)
        cpk.wait()
        cpq.wait()
        compute_chunk(c0 + 1, krb, qrb, ebb, semeb)
        return carry
    lax.fori_loop(0, (NCH - 1) // 2, body, 0)
    wait_a()
    compute_chunk(NCH - 1, kra, qra, eba, semea)
    pltpu.make_async_copy(eba, e_out.at[pl.ds(0, CH)], semea).wait()
    pltpu.make_async_copy(ebb, e_out.at[pl.ds(0, CH)], semeb).wait()
    pltpu.sync_copy(mpriv, mpart_out.at[pl.ds(wid * NP, NP)])


def _k1(k, q, src1, dst1):
    return pl.kernel(
        _k1_body,
        out_type=(jax.ShapeDtypeStruct((E,), jnp.float32),
                  jax.ShapeDtypeStruct((NW * NP,), jnp.float32)),
        mesh=_mesh,
        scratch_types=[
            pltpu.VMEM((EPW,), jnp.int32),
            pltpu.VMEM((EPW,), jnp.int32),
            pltpu.VMEM((NP,), jnp.float32),
            pltpu.VMEM((CH, D), jnp.float32),
            pltpu.VMEM((CH, D), jnp.float32),
            pltpu.VMEM((CH, D), jnp.float32),
            pltpu.VMEM((CH, D), jnp.float32),
            pltpu.VMEM((CH,), jnp.int32),
            pltpu.VMEM((CH,), jnp.int32),
            pltpu.VMEM((CH,), jnp.int32),
            pltpu.VMEM((CH,), jnp.int32),
            pltpu.VMEM((CH,), jnp.float32),
            pltpu.VMEM((CH,), jnp.float32),
            pltpu.SemaphoreType.DMA,
            pltpu.SemaphoreType.DMA,
            pltpu.SemaphoreType.DMA,
            pltpu.SemaphoreType.DMA,
            pltpu.SemaphoreType.DMA,
            pltpu.SemaphoreType.DMA,
        ],
        compiler_params=_sc_params,
    )(k, q, src1, dst1)


def _k2_body(mpart_in, m_out, acc, tmp):
    c = lax.axis_index("c")
    s = lax.axis_index("s")
    wid = s * NC + c
    base = wid * NPW
    pltpu.sync_copy(mpart_in.at[pl.ds(base, NPW)], acc)

    def red(j, carry):
        pltpu.sync_copy(mpart_in.at[pl.ds(j * NP + base, NPW)], tmp)
        for t in range(NPW // L):
            sl = pl.ds(t * L, L)
            acc[sl] = jnp.maximum(acc[sl], tmp[sl])
        return carry
    lax.fori_loop(1, NW, red, 0)
    pltpu.sync_copy(acc, m_out.at[pl.ds(base, NPW)])


def _tc_relay_body(x_ref, y_ref):
    y_ref[...] = x_ref[...] * 1.0


def _tc_relay(x):
    return pl.pallas_call(
        _tc_relay_body,
        out_shape=jax.ShapeDtypeStruct((NP,), jnp.float32),
    )(x)


def _k2(m_part):
    return pl.kernel(
        _k2_body,
        out_type=jax.ShapeDtypeStruct((NP,), jnp.float32),
        mesh=_mesh,
        scratch_types=[
            pltpu.VMEM((NPW,), jnp.float32),
            pltpu.VMEM((NPW,), jnp.float32),
        ],
        compiler_params=_sc_params,
    )(m_part)


def _k3_body(v_hbm, src_hbm, dst_hbm, e_hbm, m_hbm, zrows_hbm, zvec_hbm,
             nump_out, denp_out,
             mloc, idxs1, exa, exb, vra, vrb, isa, isb,
             idda, iddb, eba, ebb, num_s, den_s, semva, semvb):
    c = lax.axis_index("c")
    s = lax.axis_index("s")
    wid = s * NC + c

    pltpu.sync_copy(zrows_hbm, vra)
    for j in range(NR // CH):
        pltpu.sync_copy(vra, num_s.at[pl.ds(s * NR + j * CH, CH)])
    pltpu.sync_copy(zvec_hbm, mloc.at[pl.ds(0, NR)])
    pltpu.sync_copy(mloc.at[pl.ds(0, NR)], den_s.at[pl.ds(s * NR, NR)])
    pltpu.sync_copy(src_hbm.at[pl.ds(wid * EPW, EPW)], idxs1)
    pltpu.sync_copy(m_hbm, mloc)
    plsc.subcore_barrier()

    def compute_chunk(ci, vrows, exbuf, idd, ebuf):
        def group_body(g, carry2):
            dstv = idd[pl.ds(g * L, L)]
            mv = plsc.load_gather(mloc, [dstv])
            ev = ebuf[pl.ds(g * L, L)]
            ex16 = jnp.exp(ev - mv)
            exbuf[pl.ds(g * L, L)] = ex16
            for rr in range(L):
                r = g * L + rr
                exr = jnp.full((L,), ex16[rr], jnp.float32)
                for cc in range(D // L):
                    sl = pl.ds(cc * L, L)
                    vrows[r, sl] = vrows[r, sl] * exr
            return carry2
        lax.fori_loop(0, GP, group_body, 0)
        pltpu.sync_copy(vrows, num_s.at[idd], add=True)
        pltpu.sync_copy(exbuf, den_s.at[idd], add=True)

    def issue_a(ci):
        _copy_idx(idxs1, ci * CH, isa)
        base = wid * EPW + ci * CH
        pltpu.async_copy(v_hbm.at[isa], vra, semva)
        pltpu.async_copy(dst_hbm.at[pl.ds(base, CH)], idda, semva)
        pltpu.async_copy(e_hbm.at[pl.ds(base, CH)], eba, semva)

    def wait_a():
        pltpu.make_async_copy(v_hbm.at[isa], vra, semva).wait()
        pltpu.make_async_copy(dst_hbm.at[pl.ds(0, CH)], idda, semva).wait()
        pltpu.make_async_copy(e_hbm.at[pl.ds(0, CH)], eba, semva).wait()

    issue_a(0)

    def body(i, carry):
        c0 = 2 * i
        _copy_idx(idxs1, (c0 + 1) * CH, isb)
        baseb = wid * EPW + (c0 + 1) * CH
        cpv = pltpu.async_copy(v_hbm.at[isb], vrb, semvb)
        cpd = pltpu.async_copy(dst_hbm.at[pl.ds(baseb, CH)], iddb, semvb)
        cpe = pltpu.async_copy(e_hbm.at[pl.ds(baseb, CH)], ebb, semvb)
        wait_a()
        compute_chunk(c0, vra, exa, idda, eba)
        issue_a(c0 + 2)
        cpv.wait()
        cpd.wait()
        cpe.wait()
        compute_chunk(c0 + 1, vrb, exb, iddb, ebb)
        return carry
    lax.fori_loop(0, (NCH - 1) // 2, body, 0)
    wait_a()
    compute_chunk(NCH - 1, vra, exa, idda, eba)
    plsc.subcore_barrier()
    pltpu.sync_copy(num_s.at[pl.ds(s * NR, NR)],
                    nump_out.at[c, pl.ds(s * NR, NR)])
    pltpu.sync_copy(den_s.at[pl.ds(s * NR, NR)],
                    denp_out.at[pl.ds(c * NP + s * NR, NR)])


def _k3(v, src1, dst1, e1, m, zrows, zvec):
    return pl.kernel(
        _k3_body,
        out_type=(jax.ShapeDtypeStruct((NC, NP, D), jnp.float32),
                  jax.ShapeDtypeStruct((NC * NP,), jnp.float32)),
        mesh=_mesh,
        scratch_types=[
            pltpu.VMEM((NP,), jnp.float32),
            pltpu.VMEM((EPW,), jnp.int32),
            pltpu.VMEM((CH,), jnp.float32),
            pltpu.VMEM((CH,), jnp.float32),
            pltpu.VMEM((CH, D), jnp.float32),
            pltpu.VMEM((CH, D), jnp.float32),
            pltpu.VMEM((CH,), jnp.int32),
            pltpu.VMEM((CH,), jnp.int32),
            pltpu.VMEM((CH,), jnp.int32),
            pltpu.VMEM((CH,), jnp.int32),
            pltpu.VMEM((CH,), jnp.float32),
            pltpu.VMEM((CH,), jnp.float32),
            pltpu.VMEM_SHARED((NP, D), jnp.float32),
            pltpu.VMEM_SHARED((NP,), jnp.float32),
            pltpu.SemaphoreType.DMA,
            pltpu.SemaphoreType.DMA,
        ],
        compiler_params=_sc_params,
    )(v, src1, dst1, e1, m, zrows, zvec)


def _norm_body(num_ref, den_ref, h_ref):
    n = num_ref[0] + num_ref[1]
    d = den_ref[0] + den_ref[1]
    h_ref[...] = n / (d[:, None] + 1e-16)


def _norm(num_p, den_p):
    rb = 1024
    return pl.pallas_call(
        _norm_body,
        grid=(pl.cdiv(N, rb),),
        in_specs=[pl.BlockSpec((NC, rb, D), lambda i: (0, i, 0)),
                  pl.BlockSpec((NC, rb), lambda i: (0, i))],
        out_specs=pl.BlockSpec((rb, D), lambda i: (i, 0)),
        out_shape=jax.ShapeDtypeStruct((N, D), jnp.float32),
    )(num_p, den_p)


def kernel(x, edge_index, Wk, Wq, Wv):
    src1 = edge_index[0].astype(jnp.int32)
    dst1 = edge_index[1].astype(jnp.int32)
    k, q, v = _kqv(x, Wk, Wq, Wv)
    e1, m_part = _k1(k, q, src1, dst1)
    m = _tc_relay(_k2(m_part))
    zrows = jnp.zeros((CH, D), jnp.float32)
    zvec = jnp.zeros((NR,), jnp.float32)
    num_p, den_p = _k3(v, src1, dst1, e1, m, zrows, zvec)
    return _norm(num_p, den_p.reshape(NC, NP))
